# baseline (speedup 1.0000x reference)
import functools

import jax
import jax.numpy as jnp
from jax import lax
from jax.experimental import pallas as pl
from jax.experimental.pallas import tpu as pltpu
from jax.experimental.pallas import tpu_sc as plsc

_INFO = plsc.get_sparse_core_info()
_NC = _INFO.num_cores
_NS = _INFO.num_subcores
_L = _INFO.num_lanes
_NW = _NC * _NS

_BATCH = 128


def _make_gather(V, D, B, n_pad):
    half = V // 2
    b_per_w = B // _NW
    sent = B
    mesh = plsc.VectorSubcoreMesh(core_axis_name="c", subcore_axis_name="s")

    @functools.partial(
        pl.kernel,
        mesh=mesh,
        out_type=jax.ShapeDtypeStruct((B + n_pad, D), jnp.float32),
        compiler_params=pltpu.CompilerParams(use_tc_tiling_on_sc=False),
        scratch_types=[
            pltpu.VMEM((b_per_w,), jnp.int32),
            pltpu.VMEM((b_per_w,), jnp.int32),
            pltpu.VMEM((b_per_w,), jnp.int32),
            pltpu.VMEM((b_per_w,), jnp.int32),
            pltpu.VMEM((b_per_w,), jnp.int32),
            pltpu.VMEM((_BATCH,), jnp.int32),
            pltpu.VMEM((b_per_w, D), jnp.float32),
            pltpu.VMEM((b_per_w, D), jnp.float32),
            pltpu.SemaphoreType.DMA,
        ],
    )
    def gather_kernel(h0_hbm, h1_hbm, idx_hbm, out_hbm, idx_v, i0_v, i1_v,
                      p0_v, p1_v, bpos_v, buf0_v, buf1_v, sem):
        sc = lax.axis_index("c")
        t = lax.axis_index("s")
        w = t * _NC + sc
        base = w * b_per_w
        iota = lax.iota(jnp.int32, _L)

        pltpu.sync_copy(idx_hbm.at[pl.ds(base, b_per_w)], idx_v)

        def grp(g, carry):
            v = idx_v[pl.ds(g * _L, _L)]
            j = iota + (base + g * _L)
            in0 = v < half
            i0_v[pl.ds(g * _L, _L)] = jnp.minimum(v, half - 1)
            i1_v[pl.ds(g * _L, _L)] = jnp.clip(v - half, 0, half - 1)
            svec = jnp.full((_L,), sent, jnp.int32)
            p0_v[pl.ds(g * _L, _L)] = jnp.where(in0, j, svec)
            p1_v[pl.ds(g * _L, _L)] = jnp.where(in0, svec, j)
            return carry

        lax.fori_loop(0, b_per_w // _L, grp, 0)

        pltpu.async_copy(h0_hbm.at[i0_v], buf0_v, sem).wait()
        pltpu.async_copy(h1_hbm.at[i1_v], buf1_v, sem).wait()

        def scat(args):
            buf_v, p_v = args

            def batch(b, carry):
                def cp(u, carry2):
                    bpos_v[pl.ds(u * _L, _L)] = p_v[
                        pl.ds(b * _BATCH + u * _L, _L)
                    ]
                    return carry2

                lax.fori_loop(0, _BATCH // _L, cp, 0)
                pltpu.async_copy(
                    buf_v.at[pl.ds(b * _BATCH, _BATCH)],
                    out_hbm.at[bpos_v],
                    sem,
                ).wait()
                return carry

            lax.fori_loop(0, b_per_w // _BATCH, batch, 0)

        scat((buf0_v, p0_v))
        scat((buf1_v, p1_v))

    return gather_kernel


def kernel(arg, idx):
    V, D = arg.shape
    B = idx.shape[0]
    half = V // 2
    n_pad = _L
    idx32 = idx.astype(jnp.int32)
    outpad = _make_gather(V, D, B, n_pad)(arg[:half], arg[half:], idx32)
    return outpad[:B]


# speedup vs baseline: 3.9920x; 3.9920x over previous
import functools

import jax
import jax.numpy as jnp
from jax import lax
from jax.experimental import pallas as pl
from jax.experimental.pallas import tpu as pltpu
from jax.experimental.pallas import tpu_sc as plsc


def _make_gather(V, D, B):
    info = plsc.get_sparse_core_info()
    NW = info.num_cores * info.num_subcores
    NC = info.num_cores
    L = info.num_lanes
    b_per_w = B // NW
    mesh = plsc.VectorSubcoreMesh(core_axis_name="c", subcore_axis_name="s")

    @functools.partial(
        pl.kernel,
        mesh=mesh,
        out_type=jax.ShapeDtypeStruct((B, D), jnp.float32),
        scratch_types=[
            pltpu.VMEM((b_per_w,), jnp.int32),
            pltpu.VMEM((b_per_w, D), jnp.float32),
            pltpu.SemaphoreType.DMA,
        ],
    )
    def gather_kernel(table_hbm, idx_hbm, out_hbm, idx_v, out_v, sem):
        wid = lax.axis_index("s") * NC + lax.axis_index("c")
        base = wid * b_per_w
        pltpu.sync_copy(idx_hbm.at[pl.ds(base, b_per_w)], idx_v)

        def issue_group(g, carry):
            v = idx_v[pl.ds(g * L, L)]
            for j in range(L):
                row = v[j]
                pltpu.make_async_copy(
                    table_hbm.at[pl.ds(row, 1)],
                    out_v.at[pl.ds(g * L + j, 1)],
                    sem,
                ).start()
            return carry

        lax.fori_loop(0, b_per_w // L, issue_group, 0)

        pltpu.make_async_copy(
            table_hbm.at[pl.ds(0, b_per_w)], out_v, sem
        ).wait()

        pltpu.sync_copy(out_v, out_hbm.at[pl.ds(base, b_per_w)])

    return gather_kernel


def kernel(arg, idx):
    V, D = arg.shape
    B = idx.shape[0]
    gather = _make_gather(V, D, B)
    return gather(arg, idx.astype(jnp.int32))
